# Initial kernel scaffold; baseline (speedup 1.0000x reference)
import jax
import jax.numpy as jnp
from jax.experimental import pallas as pl


def kernel(x, edge_index, edge_attr, W1, b1, W2, b2, Wfc, bfc):
    raise NotImplementedError("write your pallas kernel here")


# speedup vs baseline: 7.0626x; 7.0626x over previous
import functools

import jax
import jax.numpy as jnp
from jax import lax
from jax.experimental import pallas as pl
from jax.experimental.pallas import tpu as pltpu
from jax.experimental.pallas import tpu_sc as plsc

NN = 10000
EE = 320000
DIN = 128
HH = 64
CC = 10

NC = 2
NS = 16
NW = NC * NS
LL = 16

KCH = 128
NCH = 80
EPAD = NW * NCH * KCH
STRIP = 16
WCH = 80
NWCH = NN // WCH
NND = NCH * KCH

@functools.cache
def _sc_mesh():
    return plsc.VectorSubcoreMesh(
        core_axis_name="c", subcore_axis_name="s",
        num_cores=NC, num_subcores=NS)


def _zero_vmem_rows(buf, nrows, ncols):
    z = jnp.zeros((LL,), jnp.float32)
    for r in range(nrows):
        for j in range(ncols // LL):
            buf[r, pl.ds(j * LL, LL)] = z


@functools.cache
def _make_sc_deg():
    return functools.partial(
        pl.kernel,
        mesh=_sc_mesh(),
        out_type=jax.ShapeDtypeStruct((NC * NND,), jnp.float32),
        scratch_types=[
            pltpu.VMEM((STRIP, KCH), jnp.int32),
            pltpu.VMEM((STRIP, KCH), jnp.float32),
            pltpu.VMEM((KCH,), jnp.float32),
            pltpu.VMEM_SHARED((NND,), jnp.float32),
        ],
    )(_sc_deg_body)


def _sc_deg_body(dst_hbm, ew_hbm, out_hbm, dst_v, ew_v, stage_v, acc_sh):
    c = lax.axis_index("c")
    s = lax.axis_index("s")
    wid = c * NS + s

    for j in range(KCH // LL):
        stage_v[pl.ds(j * LL, LL)] = jnp.zeros((LL,), jnp.float32)

    def zbody(k, _):
        ci = s + k * NS
        off = pl.multiple_of(ci * KCH, KCH)
        pltpu.sync_copy(stage_v, acc_sh.at[pl.ds(off, KCH)])
        return 0

    lax.fori_loop(0, NND // KCH // NS, zbody, 0)
    plsc.subcore_barrier()

    def body(ci, _):
        m = ci % STRIP

        @pl.when(m == 0)
        def _():
            base = pl.multiple_of(ci, STRIP)
            pltpu.sync_copy(dst_hbm.at[wid, pl.ds(base, STRIP)], dst_v)
            pltpu.sync_copy(ew_hbm.at[wid, pl.ds(base, STRIP)], ew_v)

        pltpu.sync_copy(ew_v.at[m], acc_sh.at[dst_v.at[m]], add=True)
        return 0

    lax.fori_loop(0, NCH, body, 0)
    plsc.subcore_barrier()

    def wbody(k, _):
        ci = s + k * NS
        off = pl.multiple_of(ci * KCH, KCH)
        oout = pl.multiple_of(c * NND + ci * KCH, KCH)
        pltpu.sync_copy(acc_sh.at[pl.ds(off, KCH)], stage_v)
        pltpu.sync_copy(stage_v, out_hbm.at[pl.ds(oout, KCH)])
        return 0

    lax.fori_loop(0, NND // KCH // NS, wbody, 0)


@functools.cache
def _make_sc_agg():
    @functools.partial(
        pl.kernel,
        mesh=_sc_mesh(),
        out_type=jax.ShapeDtypeStruct((NC, NN, DIN), jnp.float32),
        scratch_types=[
            pltpu.VMEM((STRIP, KCH), jnp.int32),
            pltpu.VMEM((STRIP, KCH), jnp.int32),
            pltpu.VMEM((STRIP, KCH), jnp.float32),
            pltpu.VMEM((KCH, DIN), jnp.float32),
            pltpu.VMEM_SHARED((NN, DIN), jnp.float32),
            pltpu.SemaphoreType.DMA,
        ],
    )
    def _sc_agg(t_hbm, src_hbm, dst_hbm, ew_hbm, out_hbm,
                src_v, dst_v, ew_v, rows_v, acc_sh, sem):
        c = lax.axis_index("c")
        s = lax.axis_index("s")
        wid = c * NS + s

        _zero_vmem_rows(rows_v, WCH, DIN)

        def zbody(k, _):
            ci = s + k * NS

            @pl.when(ci < NWCH)
            def _():
                off = pl.multiple_of(ci * WCH, WCH)
                pltpu.sync_copy(rows_v.at[pl.ds(0, WCH)],
                                acc_sh.at[pl.ds(off, WCH)])

            return 0

        lax.fori_loop(0, pl.cdiv(NWCH, NS), zbody, 0)
        plsc.subcore_barrier()

        def body(ci, _):
            m = ci % STRIP

            @pl.when(m == 0)
            def _():
                base = pl.multiple_of(ci, STRIP)
                pltpu.sync_copy(src_hbm.at[wid, pl.ds(base, STRIP)], src_v)
                pltpu.sync_copy(dst_hbm.at[wid, pl.ds(base, STRIP)], dst_v)
                pltpu.sync_copy(ew_hbm.at[wid, pl.ds(base, STRIP)], ew_v)

            pltpu.async_copy(t_hbm.at[src_v.at[m]], rows_v, sem).wait()

            def gbody(g, _):
                wv = ew_v[m, pl.ds(g * LL, LL)]
                for l in range(LL):
                    w = jnp.full((LL,), wv[l], jnp.float32)
                    e = g * LL + l
                    for j in range(DIN // LL):
                        sl = pl.ds(j * LL, LL)
                        rows_v[e, sl] = rows_v[e, sl] * w
                return 0

            lax.fori_loop(0, KCH // LL, gbody, 0)
            pltpu.sync_copy(rows_v, acc_sh.at[dst_v.at[m]], add=True)
            return 0

        lax.fori_loop(0, NCH, body, 0)
        plsc.subcore_barrier()

        def wbody(k, _):
            ci = s + k * NS

            @pl.when(ci < NWCH)
            def _():
                off = pl.multiple_of(ci * WCH, WCH)
                pltpu.sync_copy(acc_sh.at[pl.ds(off, WCH)],
                                rows_v.at[pl.ds(0, WCH)])
                pltpu.sync_copy(rows_v.at[pl.ds(0, WCH)],
                                out_hbm.at[c, pl.ds(off, WCH)])

            return 0

        lax.fori_loop(0, pl.cdiv(NWCH, NS), wbody, 0)

    return _sc_agg


RB = 1000
GRID = NN // RB


def _tc1_body(deg_ref, x_ref, w1_ref, t1_ref, t2_ref, dinv_ref):
    deg = deg_ref[0] + deg_ref[1] + 1.0
    dinv = jnp.where(deg > 0, lax.rsqrt(deg), 0.0)
    t = jnp.dot(x_ref[...], w1_ref[...], preferred_element_type=jnp.float32)
    t1_ref[...] = t
    t2_ref[...] = t * dinv
    dinv_ref[...] = dinv


def _tc1(deg_p, x, w1):
    return pl.pallas_call(
        _tc1_body,
        grid=(GRID,),
        in_specs=[
            pl.BlockSpec((NC, RB, 1), lambda i: (0, i, 0)),
            pl.BlockSpec((RB, DIN), lambda i: (i, 0)),
            pl.BlockSpec((DIN, DIN), lambda i: (0, 0)),
        ],
        out_specs=[
            pl.BlockSpec((RB, DIN), lambda i: (i, 0)),
            pl.BlockSpec((RB, DIN), lambda i: (i, 0)),
            pl.BlockSpec((RB, 1), lambda i: (i, 0)),
        ],
        out_shape=[
            jax.ShapeDtypeStruct((NN, DIN), jnp.float32),
            jax.ShapeDtypeStruct((NN, DIN), jnp.float32),
            jax.ShapeDtypeStruct((NN, 1), jnp.float32),
        ],
    )(deg_p, x, w1)


def _tc2_body(agg_ref, t1_ref, dinv_ref, b1_ref, w2_ref, u1_ref, u2_ref):
    dinv = dinv_ref[...]
    h1 = (agg_ref[0] + agg_ref[1]) * dinv + t1_ref[...] * (dinv * dinv)
    h1 = jnp.maximum(h1 + b1_ref[...], 0.0)
    u1 = jnp.dot(h1, w2_ref[...], preferred_element_type=jnp.float32)
    u1_ref[...] = u1
    u2_ref[...] = jnp.concatenate(
        [u1 * dinv, jnp.zeros_like(u1)], axis=1)


def _tc2(agg1, t1, dinv, b1, w2):
    return pl.pallas_call(
        _tc2_body,
        grid=(GRID,),
        in_specs=[
            pl.BlockSpec((NC, RB, DIN), lambda i: (0, i, 0)),
            pl.BlockSpec((RB, DIN), lambda i: (i, 0)),
            pl.BlockSpec((RB, 1), lambda i: (i, 0)),
            pl.BlockSpec((1, DIN), lambda i: (0, 0)),
            pl.BlockSpec((DIN, HH), lambda i: (0, 0)),
        ],
        out_specs=[
            pl.BlockSpec((RB, HH), lambda i: (i, 0)),
            pl.BlockSpec((RB, DIN), lambda i: (i, 0)),
        ],
        out_shape=[
            jax.ShapeDtypeStruct((NN, HH), jnp.float32),
            jax.ShapeDtypeStruct((NN, DIN), jnp.float32),
        ],
    )(agg1, t1, dinv, b1, w2)


def _tc3_body(agg_ref, u1_ref, dinv_ref, b2_ref, wfc_ref, bfc_ref,
              out_ref, acc_ref):
    i = pl.program_id(0)
    dinv = dinv_ref[...]
    agg = agg_ref[0][:, :HH] + agg_ref[1][:, :HH]
    h2 = agg * dinv + u1_ref[...] * (dinv * dinv)
    h2 = jnp.maximum(h2 + b2_ref[...], 0.0)

    rj = lax.broadcasted_iota(jnp.int32, (HH, HH * CC), 0)
    rc = lax.broadcasted_iota(jnp.int32, (HH, HH * CC), 1)
    expand = (rc // CC == rj).astype(jnp.float32)
    h2e = jnp.dot(h2, expand, preferred_element_type=jnp.float32)

    psum = jnp.sum(h2e * wfc_ref[...], axis=0, keepdims=True)

    @pl.when(i == 0)
    def _():
        acc_ref[...] = psum

    @pl.when(i > 0)
    def _():
        acc_ref[...] = acc_ref[...] + psum

    @pl.when(i == pl.num_programs(0) - 1)
    def _():
        fj = lax.broadcasted_iota(jnp.int32, (HH * CC, CC), 0)
        fc = lax.broadcasted_iota(jnp.int32, (HH * CC, CC), 1)
        fold = (fj % CC == fc).astype(jnp.float32)
        logits = jnp.dot(acc_ref[...], fold,
                         preferred_element_type=jnp.float32) + bfc_ref[...]
        m = jnp.max(logits, axis=1, keepdims=True)
        e = jnp.exp(logits - m)
        out_ref[...] = e / jnp.sum(e, axis=1, keepdims=True)


def _tc3(agg_p, u1, dinv, b2, wfc_r, bfc):
    return pl.pallas_call(
        _tc3_body,
        grid=(GRID,),
        in_specs=[
            pl.BlockSpec((NC, RB, DIN), lambda i: (0, i, 0)),
            pl.BlockSpec((RB, HH), lambda i: (i, 0)),
            pl.BlockSpec((RB, 1), lambda i: (i, 0)),
            pl.BlockSpec((1, HH), lambda i: (0, 0)),
            pl.BlockSpec((RB, HH * CC), lambda i: (i, 0)),
            pl.BlockSpec((1, CC), lambda i: (0, 0)),
        ],
        out_specs=pl.BlockSpec((1, CC), lambda i: (0, 0)),
        out_shape=jax.ShapeDtypeStruct((1, CC), jnp.float32),
        scratch_shapes=[pltpu.VMEM((1, HH * CC), jnp.float32)],
    )(agg_p, u1, dinv, b2, wfc_r, bfc)


@jax.jit
def kernel(x, edge_index, edge_attr, W1, b1, W2, b2, Wfc, bfc):
    pad = EPAD - EE
    src3 = jnp.concatenate(
        [edge_index[0], jnp.zeros((pad,), edge_index.dtype)]
    ).reshape(NW, NCH, KCH)
    dst3 = jnp.concatenate(
        [edge_index[1], jnp.zeros((pad,), edge_index.dtype)]
    ).reshape(NW, NCH, KCH)
    ew3 = jnp.concatenate(
        [edge_attr, jnp.zeros((pad,), edge_attr.dtype)]
    ).reshape(NW, NCH, KCH)

    deg_p = _make_sc_deg()(dst3, ew3)
    deg_p = deg_p.reshape(NC, NND, 1)[:, :NN]
    t1, t2, dinv = _tc1(deg_p, x, W1)
    agg1 = _make_sc_agg()(t2, src3, dst3, ew3)
    u1, u2 = _tc2(agg1, t1, dinv, b1.reshape(1, DIN), W2)
    agg2 = _make_sc_agg()(u2, src3, dst3, ew3)
    return _tc3(agg2, u1, dinv, b2.reshape(1, HH),
                Wfc.reshape(NN, HH * CC), bfc.reshape(1, CC))
